# baseline (speedup 1.0000x reference)
import functools

import jax
import jax.numpy as jnp
from jax import lax
from jax.experimental import pallas as pl
from jax.experimental.pallas import tpu as pltpu
from jax.experimental.pallas import tpu_sc as plsc

D = 128
NC, NS = 2, 16
NW = NC * NS
CHUNK = 128


def _sc_gather(ids2, tab):
    n_chunks = ids2.shape[0]
    B = n_chunks * CHUNK
    kpw = n_chunks // NW
    mesh = plsc.VectorSubcoreMesh(
        core_axis_name="c", subcore_axis_name="s",
        num_cores=NC, num_subcores=NS)

    @functools.partial(
        pl.kernel,
        out_type=jax.ShapeDtypeStruct((B, D), jnp.float32),
        mesh=mesh,
        scratch_types=(
            [pltpu.VMEM((kpw, CHUNK), jnp.int32)]
            + [pltpu.VMEM((CHUNK, D), jnp.float32) for _ in range(kpw)]
            + [pltpu.SemaphoreType.DMA for _ in range(2 * kpw)]
        ),
    )
    def k(ids_hbm, tab_hbm, out_hbm, idx_v, *rest):
        bufs = rest[:kpw]
        gsem = rest[kpw:2 * kpw]
        wsem = rest[2 * kpw:]
        wid = lax.axis_index("s") * NC + lax.axis_index("c")
        cbase = wid * kpw
        pltpu.sync_copy(ids_hbm.at[pl.ds(cbase, kpw)], idx_v)
        gathers = [
            pltpu.async_copy(tab_hbm.at[idx_v.at[j]], bufs[j], gsem[j])
            for j in range(kpw)
        ]
        writes = []
        for j in range(kpw):
            gathers[j].wait()
            writes.append(pltpu.async_copy(
                bufs[j], out_hbm.at[pl.ds((cbase + j) * CHUNK, CHUNK)],
                wsem[j]))
        for w in writes:
            w.wait()

    return k(ids2, tab)


def _tc_project(x, W, b):
    B = x.shape[0]
    BM = 2048
    dn = (((1,), (1,)), ((), ()))

    def body(x_ref, w_ref, b_ref, o_ref):
        o_ref[...] = lax.dot_general(
            x_ref[...], w_ref[...], dn,
            preferred_element_type=jnp.float32) + b_ref[...]

    return pl.pallas_call(
        body,
        grid=(B // BM,),
        in_specs=[
            pl.BlockSpec((BM, D), lambda i: (i, 0)),
            pl.BlockSpec((D, D), lambda i: (0, 0)),
            pl.BlockSpec((1, D), lambda i: (0, 0)),
        ],
        out_specs=pl.BlockSpec((BM, D), lambda i: (i, 0)),
        out_shape=jax.ShapeDtypeStruct((B, D), jnp.float32),
    )(x, W, b.reshape(1, D))


def kernel(user_ids, item_ids, user_embedding, item_embedding,
           W_user, b_user, W_item, b_item):
    B = user_ids.shape[0]
    uids2 = user_ids.astype(jnp.int32).reshape(B // CHUNK, CHUNK)
    iids2 = item_ids.astype(jnp.int32).reshape(B // CHUNK, CHUNK)
    gu = _sc_gather(uids2, user_embedding)
    gi = _sc_gather(iids2, item_embedding)
    ou = _tc_project(gu, W_user, b_user)
    oi = _tc_project(gi, W_item, b_item)
    return (ou, oi)
